# baseline (speedup 1.0000x reference)
import functools

import jax
import jax.numpy as jnp
from jax import lax
from jax.experimental import pallas as pl
from jax.experimental.pallas import tpu as pltpu
from jax.experimental.pallas import tpu_sc as plsc

_NC = 2
_NS = 16
_NW = _NC * _NS
_EP = 128
_G = 2048
_TB1 = 2048
_TB2 = 1024


def _make_sc_gather(G):
    t_pw = G // _NW
    n_tr = max(1, t_pw // 128)
    per = t_pw // n_tr
    mesh = plsc.VectorSubcoreMesh(
        core_axis_name="c", subcore_axis_name="s",
        num_cores=_NC, num_subcores=_NS,
    )

    @functools.partial(
        pl.kernel,
        out_type=jax.ShapeDtypeStruct((G, _EP), jnp.float32),
        mesh=mesh,
        scratch_types=[
            pltpu.VMEM((t_pw,), jnp.int32),
            pltpu.VMEM((t_pw, _EP), jnp.float32),
            pltpu.SemaphoreType.DMA,
        ],
    )
    def gather(ids_hbm, emb_hbm, out_hbm, idx_v, rows_v, sem):
        wid = lax.axis_index("s") * _NC + lax.axis_index("c")
        base = wid * t_pw
        pltpu.sync_copy(ids_hbm.at[pl.ds(base, t_pw)], idx_v)
        handles = [
            pltpu.async_copy(
                emb_hbm.at[idx_v.at[pl.ds(c * per, per)]],
                rows_v.at[pl.ds(c * per, per)],
                sem,
            )
            for c in range(n_tr)
        ]
        for h in handles:
            h.wait()
        pltpu.sync_copy(rows_v, out_hbm.at[pl.ds(base, t_pw)])

    return gather


def _onehot_body(V, TB, ids_ref, emb_ref, w_ref, o_ref):
    ids = ids_ref[0, 0, :]
    iota_v = lax.broadcasted_iota(jnp.int32, (V, TB), 0)
    eq = iota_v == lax.broadcast_in_dim(ids, (V, TB), (1,))
    oh_t = eq.astype(jnp.float32).astype(jnp.bfloat16)
    h_t = lax.dot_general(
        emb_ref[...], oh_t,
        dimension_numbers=(((0,), (0,)), ((), ())),
        preferred_element_type=jnp.float32,
    )
    o_ref[...] = lax.dot_general(
        w_ref[...], h_t[:6, :],
        dimension_numbers=(((1,), (0,)), ((), ())),
        preferred_element_type=jnp.float32,
    )


def _tc1_onehot_proj(ids_hi, emb_bf16, W6, V, T, G):
    nb = (T - G) // _TB1
    return pl.pallas_call(
        functools.partial(_onehot_body, V, _TB1),
        grid=(nb,),
        in_specs=[
            pl.BlockSpec((1, 1, _TB1), lambda i: (i, 0, 0)),
            pl.BlockSpec((V, _EP), lambda i: (0, 0)),
            pl.BlockSpec((V, 6), lambda i: (0, 0)),
        ],
        out_specs=pl.BlockSpec((V, _TB1), lambda i: (0, i + G // _TB1)),
        out_shape=jax.ShapeDtypeStruct((V, T), jnp.float32),
    )(ids_hi.reshape(nb, 1, _TB1), emb_bf16, W6)


def _tc2_body(prev_ref, w_ref, h_ref, o_ref):
    h6 = h_ref[...][:, :6]
    o_ref[...] = lax.dot_general(
        w_ref[...], h6,
        dimension_numbers=(((1,), (1,)), ((), ())),
        preferred_element_type=jnp.float32,
    )


def _tc2_fill(prev, W6, hidden_lo, V, T, G):
    return pl.pallas_call(
        _tc2_body,
        grid=(G // _TB2,),
        in_specs=[
            pl.BlockSpec(memory_space=pl.ANY),
            pl.BlockSpec((V, 6), lambda i: (0, 0)),
            pl.BlockSpec((_TB2, _EP), lambda i: (i, 0)),
        ],
        out_specs=pl.BlockSpec((V, _TB2), lambda i: (0, i)),
        out_shape=jax.ShapeDtypeStruct((V, T), jnp.float32),
        input_output_aliases={0: 0},
    )(prev, W6, hidden_lo)


def kernel(input_ids, cu_seq_lens_q, cu_seq_lens_k, max_length_q,
           max_length_k, position_ids, text_position_ids, pack_num_samples,
           embed_table, W, b):
    B, T = input_ids.shape
    V, D = embed_table.shape
    T = B * T
    ids = input_ids.reshape(-1).astype(jnp.int32)
    emb_pad = jnp.concatenate(
        [embed_table, jnp.ones((V, 1), jnp.float32),
         jnp.zeros((V, _EP - D - 1), jnp.float32)], axis=1)
    W6 = jnp.concatenate([W, b.reshape(V, 1)], axis=1)
    hidden_lo = _make_sc_gather(_G)(ids[:_G], emb_pad)
    out_t = _tc1_onehot_proj(ids[_G:], emb_pad.astype(jnp.bfloat16), W6,
                             V, T, _G)
    out_t = _tc2_fill(out_t, W6, hidden_lo, V, T, _G)
    return jnp.transpose(out_t).reshape(B, input_ids.shape[1], V)


# speedup vs baseline: 1.0329x; 1.0329x over previous
import functools

import jax
import jax.numpy as jnp
from jax import lax
from jax.experimental import pallas as pl
from jax.experimental.pallas import tpu as pltpu
from jax.experimental.pallas import tpu_sc as plsc

_NC = 2
_NS = 16
_NW = _NC * _NS
_EP = 128
_G = 4096
_TB1 = 2048
_TB2 = 2048


def _make_sc_gather(G, T):
    t_pw = G // _NW
    n_tr = max(1, t_pw // 128)
    per = t_pw // n_tr
    mesh = plsc.VectorSubcoreMesh(
        core_axis_name="c", subcore_axis_name="s",
        num_cores=_NC, num_subcores=_NS,
    )

    @functools.partial(
        pl.kernel,
        out_type=jax.ShapeDtypeStruct((G, _EP), jnp.float32),
        mesh=mesh,
        scratch_types=[
            pltpu.VMEM((t_pw,), jnp.int32),
            pltpu.VMEM((t_pw, _EP), jnp.float32),
            pltpu.SemaphoreType.DMA,
        ],
    )
    def gather(ids_hbm, emb_hbm, out_hbm, idx_v, rows_v, sem):
        wid = lax.axis_index("s") * _NC + lax.axis_index("c")
        base = wid * t_pw
        pltpu.sync_copy(ids_hbm.at[pl.ds(base, t_pw)], idx_v)
        handles = [
            pltpu.async_copy(
                emb_hbm.at[idx_v.at[pl.ds(c * per, per)]],
                rows_v.at[pl.ds(c * per, per)],
                sem,
            )
            for c in range(n_tr)
        ]
        for h in handles:
            h.wait()
        pltpu.sync_copy(rows_v, out_hbm.at[pl.ds(base, t_pw)])

    return gather


def _onehot_body(V, TB, ids_ref, emb_ref, w_ref, o_ref):
    ids = ids_ref[0, 0, :]
    iota_v = lax.broadcasted_iota(jnp.int32, (V, TB), 0)
    eq = iota_v == lax.broadcast_in_dim(ids, (V, TB), (1,))
    oh_t = eq.astype(jnp.float32).astype(jnp.bfloat16)
    h_t = lax.dot_general(
        emb_ref[...], oh_t,
        dimension_numbers=(((0,), (0,)), ((), ())),
        preferred_element_type=jnp.float32,
    )
    o_ref[...] = lax.dot_general(
        w_ref[...], h_t[:6, :],
        dimension_numbers=(((1,), (0,)), ((), ())),
        preferred_element_type=jnp.float32,
    )


def _tc1_onehot_proj(ids3, emb_bf16, W6, V, T, G):
    nb = (T - G) // _TB1
    off = G // _TB1
    return pl.pallas_call(
        functools.partial(_onehot_body, V, _TB1),
        grid=(nb,),
        in_specs=[
            pl.BlockSpec((1, 1, _TB1), lambda i: (i + off, 0, 0)),
            pl.BlockSpec((V, _EP), lambda i: (0, 0)),
            pl.BlockSpec((V, 6), lambda i: (0, 0)),
        ],
        out_specs=pl.BlockSpec((V, _TB1), lambda i: (0, i + off)),
        out_shape=jax.ShapeDtypeStruct((V, T), jnp.float32),
    )(ids3, emb_bf16, W6)


def _tc2_body(prev_ref, w_ref, h_ref, o_ref):
    h6 = h_ref[...][:, :6]
    o_ref[...] = lax.dot_general(
        w_ref[...], h6,
        dimension_numbers=(((1,), (1,)), ((), ())),
        preferred_element_type=jnp.float32,
    )


def _tc2_fill(prev, W6, hidden_lo, V, T, G):
    return pl.pallas_call(
        _tc2_body,
        grid=(G // _TB2,),
        in_specs=[
            pl.BlockSpec(memory_space=pl.ANY),
            pl.BlockSpec((V, 6), lambda i: (0, 0)),
            pl.BlockSpec((_TB2, _EP), lambda i: (i, 0)),
        ],
        out_specs=pl.BlockSpec((V, _TB2), lambda i: (0, i)),
        out_shape=jax.ShapeDtypeStruct((V, T), jnp.float32),
        input_output_aliases={0: 0},
    )(prev, W6, hidden_lo)


def kernel(input_ids, cu_seq_lens_q, cu_seq_lens_k, max_length_q,
           max_length_k, position_ids, text_position_ids, pack_num_samples,
           embed_table, W, b):
    B, T0 = input_ids.shape
    V, D = embed_table.shape
    T = B * T0
    ids = input_ids.reshape(-1).astype(jnp.int32)
    ids3 = ids.reshape(T // _TB1, 1, _TB1)
    emb_pad = jnp.concatenate(
        [embed_table, jnp.ones((V, 1), jnp.float32),
         jnp.zeros((V, _EP - D - 1), jnp.float32)], axis=1)
    W6 = jnp.concatenate([W, b.reshape(V, 1)], axis=1)
    hidden_lo = _make_sc_gather(_G, T)(ids, emb_pad)
    out_t = _tc1_onehot_proj(ids3, emb_pad.astype(jnp.bfloat16), W6,
                             V, T, _G)
    out_t = _tc2_fill(out_t, W6, hidden_lo, V, T, _G)
    return jnp.transpose(out_t).reshape(B, T0, V)
